# baseline (speedup 1.0000x reference)
import jax
import jax.numpy as jnp
from jax import lax
from jax.experimental import pallas as pl
from jax.experimental.pallas import tpu as pltpu
from jax.experimental.pallas import tpu_sc as plsc

VOCAB = 1000000
EMBED = 64
BATCH = 16384
SUBWORDS = 50

NUM_CORES = 2
NUM_SUBCORES = 16
NUM_WORKERS = NUM_CORES * NUM_SUBCORES
ROWS_PER_WORKER = BATCH // NUM_WORKERS
SLAB_ROWS = 1
SLABS = ROWS_PER_WORKER // SLAB_ROWS
GATHER_PER_SLAB = SLAB_ROWS * SUBWORDS
IDX_MINOR = 50
IDX_MAJOR = GATHER_PER_SLAB // IDX_MINOR
IDX_ROWS_PER_WORKER = SLABS * IDX_MAJOR
LANES = 16
VREGS_PER_ROW = EMBED // LANES
NBUF = 16


def _body(idx_hbm, table_hbm, out_hbm, idx_v, gbuf, acc_out, sem0, sem1,
          sem2, sem3, sem4, sem5, sem6, sem7, sem8, sem9, sem10, sem11,
          sem12, sem13, sem14, sem15, sem_out):
    c = lax.axis_index("c")
    s_ax = lax.axis_index("s")
    wid = s_ax * NUM_CORES + c
    sems = (sem0, sem1, sem2, sem3, sem4, sem5, sem6, sem7,
            sem8, sem9, sem10, sem11, sem12, sem13, sem14, sem15)

    pltpu.sync_copy(
        idx_hbm.at[pl.ds(wid * IDX_ROWS_PER_WORKER, IDX_ROWS_PER_WORKER)],
        idx_v,
    )

    def issue(s, b):
        for r in range(IDX_MAJOR):
            pltpu.async_copy(
                table_hbm.at[idx_v.at[s * IDX_MAJOR + r]],
                gbuf.at[b, pl.ds(r * IDX_MINOR, IDX_MINOR)],
                sems[b],
            )

    def drain(s, b):
        for r in range(IDX_MAJOR):
            pltpu.make_async_copy(
                table_hbm.at[idx_v.at[s * IDX_MAJOR + r]],
                gbuf.at[b, pl.ds(r * IDX_MINOR, IDX_MINOR)],
                sems[b],
            ).wait()

    def accumulate(s, b):
        for r in range(SLAB_ROWS):
            def j_body(j, accs, _r=r):
                row = _r * SUBWORDS + j
                return tuple(
                    accs[v] + gbuf[b, row, pl.ds(v * LANES, LANES)]
                    for v in range(VREGS_PER_ROW)
                )
            accs = lax.fori_loop(
                0, SUBWORDS, j_body,
                tuple(jnp.zeros((LANES,), jnp.float32)
                      for _ in range(VREGS_PER_ROW)),
                unroll=10,
            )
            for v in range(VREGS_PER_ROW):
                acc_out[s * SLAB_ROWS + r, pl.ds(v * LANES, LANES)] = accs[v]
        pltpu.async_copy(
            acc_out.at[pl.ds(s * SLAB_ROWS, SLAB_ROWS)],
            out_hbm.at[pl.ds(wid * ROWS_PER_WORKER + s * SLAB_ROWS,
                             SLAB_ROWS)],
            sem_out,
        )

    for b in range(NBUF):
        issue(b, b)

    def pair_body(s2, carry):
        for b in range(NBUF):
            s = s2 * NBUF + b
            drain(s, b)
            accumulate(s, b)
            issue(s + NBUF, b)
        return carry

    lax.fori_loop(0, SLABS // NBUF - 1, pair_body, 0)

    for b in range(NBUF):
        s = SLABS - NBUF + b
        drain(s, b)
        accumulate(s, b)

    pltpu.make_async_copy(
        out_hbm.at[pl.ds(wid * ROWS_PER_WORKER, ROWS_PER_WORKER)],
        acc_out,
        sem_out,
    ).wait()


def kernel(xinput, table):
    idx = xinput.astype(jnp.int32).reshape(
        BATCH * SUBWORDS // IDX_MINOR, IDX_MINOR)
    mesh = plsc.VectorSubcoreMesh(core_axis_name="c", subcore_axis_name="s")
    f = pl.kernel(
        _body,
        out_type=jax.ShapeDtypeStruct((BATCH, EMBED), jnp.float32),
        mesh=mesh,
        scratch_types=[
            pltpu.VMEM((IDX_ROWS_PER_WORKER, IDX_MINOR), jnp.int32),
            pltpu.VMEM((NBUF, GATHER_PER_SLAB, EMBED), jnp.float32),
            pltpu.VMEM((ROWS_PER_WORKER, EMBED), jnp.float32),
            pltpu.SemaphoreType.DMA,
            pltpu.SemaphoreType.DMA,
            pltpu.SemaphoreType.DMA,
            pltpu.SemaphoreType.DMA,
            pltpu.SemaphoreType.DMA,
            pltpu.SemaphoreType.DMA,
            pltpu.SemaphoreType.DMA,
            pltpu.SemaphoreType.DMA,
            pltpu.SemaphoreType.DMA,
            pltpu.SemaphoreType.DMA,
            pltpu.SemaphoreType.DMA,
            pltpu.SemaphoreType.DMA,
            pltpu.SemaphoreType.DMA,
            pltpu.SemaphoreType.DMA,
            pltpu.SemaphoreType.DMA,
            pltpu.SemaphoreType.DMA,
            pltpu.SemaphoreType.DMA,
        ],
        compiler_params=pltpu.CompilerParams(use_tc_tiling_on_sc=False),
    )
    return f(idx, table)


# speedup vs baseline: 1.0374x; 1.0374x over previous
import jax
import jax.numpy as jnp
from jax import lax
from jax.experimental import pallas as pl
from jax.experimental.pallas import tpu as pltpu
from jax.experimental.pallas import tpu_sc as plsc

VOCAB = 1000000
EMBED = 64
BATCH = 16384
SUBWORDS = 50

NUM_CORES = 2
NUM_SUBCORES = 16
NUM_WORKERS = NUM_CORES * NUM_SUBCORES
ROWS_PER_WORKER = BATCH // NUM_WORKERS
SLAB_ROWS = 2
SLABS = ROWS_PER_WORKER // SLAB_ROWS
GATHER_PER_SLAB = SLAB_ROWS * SUBWORDS
IDX_MINOR = 100
IDX_MAJOR = GATHER_PER_SLAB // IDX_MINOR
IDX_ROWS_PER_WORKER = SLABS * IDX_MAJOR
LANES = 16
VREGS_PER_ROW = EMBED // LANES
NBUF = 8


def _body(idx_hbm, table_hbm, out_hbm, idx_v, gbuf, acc_out, sem0, sem1,
          sem2, sem3, sem4, sem5, sem6, sem7, sem_out):
    c = lax.axis_index("c")
    s_ax = lax.axis_index("s")
    wid = s_ax * NUM_CORES + c
    sems = (sem0, sem1, sem2, sem3, sem4, sem5, sem6, sem7)

    pltpu.sync_copy(
        idx_hbm.at[pl.ds(wid * IDX_ROWS_PER_WORKER, IDX_ROWS_PER_WORKER)],
        idx_v,
    )

    def issue(s, b):
        for r in range(IDX_MAJOR):
            pltpu.async_copy(
                table_hbm.at[idx_v.at[s * IDX_MAJOR + r]],
                gbuf.at[b, pl.ds(r * IDX_MINOR, IDX_MINOR)],
                sems[b],
            )

    def drain(s, b):
        for r in range(IDX_MAJOR):
            pltpu.make_async_copy(
                table_hbm.at[idx_v.at[s * IDX_MAJOR + r]],
                gbuf.at[b, pl.ds(r * IDX_MINOR, IDX_MINOR)],
                sems[b],
            ).wait()

    def accumulate(s, b):
        for r in range(SLAB_ROWS):
            def j_body(j, accs, _r=r):
                row = _r * SUBWORDS + j
                return tuple(
                    accs[v] + gbuf[b, row, pl.ds(v * LANES, LANES)]
                    for v in range(VREGS_PER_ROW)
                )
            accs = lax.fori_loop(
                0, SUBWORDS, j_body,
                tuple(jnp.zeros((LANES,), jnp.float32)
                      for _ in range(VREGS_PER_ROW)),
                unroll=10,
            )
            for v in range(VREGS_PER_ROW):
                acc_out[s * SLAB_ROWS + r, pl.ds(v * LANES, LANES)] = accs[v]
        pltpu.async_copy(
            acc_out.at[pl.ds(s * SLAB_ROWS, SLAB_ROWS)],
            out_hbm.at[pl.ds(wid * ROWS_PER_WORKER + s * SLAB_ROWS,
                             SLAB_ROWS)],
            sem_out,
        )

    for b in range(NBUF):
        issue(b, b)

    def pair_body(s2, carry):
        for b in range(NBUF):
            s = s2 * NBUF + b
            drain(s, b)
            accumulate(s, b)
            issue(s + NBUF, b)
        return carry

    lax.fori_loop(0, SLABS // NBUF - 1, pair_body, 0)

    for b in range(NBUF):
        s = SLABS - NBUF + b
        drain(s, b)
        accumulate(s, b)

    pltpu.make_async_copy(
        out_hbm.at[pl.ds(wid * ROWS_PER_WORKER, ROWS_PER_WORKER)],
        acc_out,
        sem_out,
    ).wait()


def kernel(xinput, table):
    idx = xinput.astype(jnp.int32).reshape(
        BATCH * SUBWORDS // IDX_MINOR, IDX_MINOR)
    mesh = plsc.VectorSubcoreMesh(core_axis_name="c", subcore_axis_name="s")
    f = pl.kernel(
        _body,
        out_type=jax.ShapeDtypeStruct((BATCH, EMBED), jnp.float32),
        mesh=mesh,
        scratch_types=[
            pltpu.VMEM((IDX_ROWS_PER_WORKER, IDX_MINOR), jnp.int32),
            pltpu.VMEM((NBUF, GATHER_PER_SLAB, EMBED), jnp.float32),
            pltpu.VMEM((ROWS_PER_WORKER, EMBED), jnp.float32),
            pltpu.SemaphoreType.DMA,
            pltpu.SemaphoreType.DMA,
            pltpu.SemaphoreType.DMA,
            pltpu.SemaphoreType.DMA,
            pltpu.SemaphoreType.DMA,
            pltpu.SemaphoreType.DMA,
            pltpu.SemaphoreType.DMA,
            pltpu.SemaphoreType.DMA,
            pltpu.SemaphoreType.DMA,
        ],
        compiler_params=pltpu.CompilerParams(use_tc_tiling_on_sc=False),
    )
    return f(idx, table)
